# Initial kernel scaffold; baseline (speedup 1.0000x reference)
import jax
import jax.numpy as jnp
from jax.experimental import pallas as pl


def kernel(x, edge_index, batch, W_gcn, b_gcn, W_sage_l, b_sage_l, W_sage_r, W_fc1, b_fc1, W_fc2, b_fc2):
    raise NotImplementedError("write your pallas kernel here")


# speedup vs baseline: 7.1472x; 7.1472x over previous
import functools

import jax
import jax.numpy as jnp
from jax import lax
from jax.experimental import pallas as pl
from jax.experimental.pallas import tpu as pltpu
from jax.experimental.pallas import tpu_sc as plsc

N = 10000
E = 320000
F = 128
G = 32
OUT = 3

NC = 2
NS = 16
NW = NC * NS
CH = 128
NCH = 80
EPT = CH * NCH
E_PAD = EPT * NW
N_PAD = 10240
ROWS_PER_TILE = N_PAD // NS

RB = 2000
NRB = N // RB


def _fill2d(ref, nrows, ncols, value):
    v = jnp.full((16,), value, dtype=jnp.float32)

    def body(r, carry):
        for j in range(ncols // 16):
            ref[r, pl.ds(j * 16, 16)] = v
        return carry

    lax.fori_loop(0, nrows, body, 0)


_MESH = plsc.VectorSubcoreMesh(
    core_axis_name="c", subcore_axis_name="s", num_cores=NC, num_subcores=NS
)


@functools.partial(
    pl.kernel,
    out_type=jax.ShapeDtypeStruct((NC, N_PAD, F), jnp.float32),
    mesh=_MESH,
    scratch_types=[
        pltpu.VMEM((NCH, CH), jnp.int32),
        pltpu.VMEM((CH, F), jnp.float32),
        pltpu.VMEM_SHARED((N_PAD, F), jnp.float32),
    ],
)
def _count_kernel(dst_hbm, out_hbm, dst_v, ones_v, acc):
    cid = lax.axis_index("c")
    sid = lax.axis_index("s")
    wid = cid * NS + sid
    base = sid * ROWS_PER_TILE

    pltpu.sync_copy(dst_hbm.at[wid], dst_v)

    _fill2d(ones_v, CH, F, 0.0)
    for k in range(ROWS_PER_TILE // CH):
        pltpu.sync_copy(ones_v, acc.at[pl.ds(base + k * CH, CH)])
    _fill2d(ones_v, CH, F, 1.0)
    plsc.subcore_barrier()

    def body(c, carry):
        pltpu.sync_copy(ones_v, acc.at[dst_v.at[c]], add=True)
        return carry

    lax.fori_loop(0, NCH, body, 0)

    plsc.subcore_barrier()
    pltpu.sync_copy(acc.at[pl.ds(base, ROWS_PER_TILE)],
                    out_hbm.at[cid, pl.ds(base, ROWS_PER_TILE)])


NCH_H = NCH // 2


@functools.partial(
    pl.kernel,
    out_type=jax.ShapeDtypeStruct((NC, N_PAD, F), jnp.float32),
    mesh=_MESH,
    scratch_types=[
        pltpu.VMEM((NCH_H, CH), jnp.int32),
        pltpu.VMEM((NCH_H, CH), jnp.int32),
        pltpu.VMEM((CH, F), jnp.float32),
        pltpu.VMEM((CH, F), jnp.float32),
        pltpu.SemaphoreType.DMA,
        pltpu.SemaphoreType.DMA,
        pltpu.VMEM_SHARED((N_PAD, F), jnp.float32),
    ],
)
def _agg_kernel(src_hbm, dst_hbm, val_hbm, out_hbm,
                src_v, dst_v, buf0, buf1, sem0, sem1, acc):
    cid = lax.axis_index("c")
    sid = lax.axis_index("s")
    wid = cid * NS + sid
    base = sid * ROWS_PER_TILE

    _fill2d(buf0, CH, F, 0.0)
    for k in range(ROWS_PER_TILE // CH):
        pltpu.sync_copy(buf0, acc.at[pl.ds(base + k * CH, CH)])
    plsc.subcore_barrier()

    for h in range(NCH // NCH_H):
        pltpu.sync_copy(src_hbm.at[wid, pl.ds(h * NCH_H, NCH_H)], src_v)
        pltpu.sync_copy(dst_hbm.at[wid, pl.ds(h * NCH_H, NCH_H)], dst_v)

        pltpu.async_copy(val_hbm.at[src_v.at[0]], buf0, sem0)
        pltpu.async_copy(val_hbm.at[src_v.at[1]], buf1, sem1)

        def body(g, carry):
            c0 = 2 * g
            pltpu.make_async_copy(val_hbm.at[pl.ds(0, CH)], buf0, sem0).wait()
            pltpu.sync_copy(buf0, acc.at[dst_v.at[c0]], add=True)
            pltpu.async_copy(val_hbm.at[src_v.at[c0 + 2]], buf0, sem0)
            pltpu.make_async_copy(val_hbm.at[pl.ds(0, CH)], buf1, sem1).wait()
            pltpu.sync_copy(buf1, acc.at[dst_v.at[c0 + 1]], add=True)
            pltpu.async_copy(val_hbm.at[src_v.at[c0 + 3]], buf1, sem1)
            return carry

        lax.fori_loop(0, NCH_H // 2 - 1, body, 0)

        pltpu.make_async_copy(val_hbm.at[pl.ds(0, CH)], buf0, sem0).wait()
        pltpu.sync_copy(buf0, acc.at[dst_v.at[NCH_H - 2]], add=True)
        pltpu.make_async_copy(val_hbm.at[pl.ds(0, CH)], buf1, sem1).wait()
        pltpu.sync_copy(buf1, acc.at[dst_v.at[NCH_H - 1]], add=True)

    plsc.subcore_barrier()
    pltpu.sync_copy(acc.at[pl.ds(base, ROWS_PER_TILE)],
                    out_hbm.at[cid, pl.ds(base, ROWS_PER_TILE)])


def _k2_body(x_ref, w_ref, cntp_ref, y_ref, dinv_ref, invc_ref):
    cnt = cntp_ref[0, :, 0:1] + cntp_ref[1, :, 0:1]
    dinv = lax.rsqrt(cnt + 1.0)
    xw = jnp.dot(x_ref[...], w_ref[...], preferred_element_type=jnp.float32)
    y_ref[...] = xw * dinv
    dinv_ref[...] = dinv
    invc_ref[...] = 1.0 / jnp.maximum(cnt, 1.0)


def _k4_body(a1_ref, y_ref, dinv_ref, bg_ref, wr_ref, bl_ref, h1_ref, r_ref):
    h1 = jax.nn.relu((a1_ref[0] + a1_ref[1] + y_ref[...]) * dinv_ref[...]
                     + bg_ref[...])
    h1_ref[...] = h1
    r_ref[...] = (jnp.dot(h1, wr_ref[...], preferred_element_type=jnp.float32)
                  + bl_ref[...])


def _k6_body(a2_ref, invc_ref, r_ref, wl_ref, batch_ref,
             wf1_ref, bf1_ref, wf2_ref, bf2_ref, out_ref, acc_ref):
    i = pl.program_id(0)
    mean = (a2_ref[0] + a2_ref[1]) * invc_ref[...]
    h2 = jax.nn.relu(
        jnp.dot(mean, wl_ref[...], preferred_element_type=jnp.float32)
        + r_ref[...])
    seg = batch_ref[0]
    onehot = (lax.broadcasted_iota(jnp.int32, (G, RB), 0) == seg
              ).astype(jnp.float32)
    contrib = jnp.dot(onehot, h2, preferred_element_type=jnp.float32)

    @pl.when(i == 0)
    def _():
        acc_ref[...] = jnp.zeros_like(acc_ref)

    acc_ref[...] += contrib

    @pl.when(i == NRB - 1)
    def _():
        z = jax.nn.relu(
            jnp.dot(acc_ref[...], wf1_ref[...],
                    preferred_element_type=jnp.float32) + bf1_ref[...])
        out_ref[...] = (jnp.dot(z, wf2_ref[...],
                                preferred_element_type=jnp.float32)
                        + bf2_ref[...])


def kernel(x, edge_index, batch, W_gcn, b_gcn, W_sage_l, b_sage_l, W_sage_r,
           W_fc1, b_fc1, W_fc2, b_fc2):
    pad = E_PAD - E
    src_r = jnp.concatenate(
        [edge_index[0], jnp.zeros((pad,), jnp.int32)]).reshape(NW, NCH, CH)
    dst_r = jnp.concatenate(
        [edge_index[1], jnp.full((pad,), N, jnp.int32)]).reshape(NW, NCH, CH)
    batch3d = batch.reshape(NRB, 1, RB)
    bg = b_gcn.reshape(1, F)
    bl = b_sage_l.reshape(1, F)
    bf1 = b_fc1.reshape(1, F)
    bf2 = b_fc2.reshape(1, OUT)

    cntp = _count_kernel(dst_r)[:, :N, :]

    y, dinv, invc = pl.pallas_call(
        _k2_body,
        grid=(NRB,),
        in_specs=[
            pl.BlockSpec((RB, F), lambda i: (i, 0)),
            pl.BlockSpec((F, F), lambda i: (0, 0)),
            pl.BlockSpec((NC, RB, F), lambda i: (0, i, 0)),
        ],
        out_specs=[
            pl.BlockSpec((RB, F), lambda i: (i, 0)),
            pl.BlockSpec((RB, 1), lambda i: (i, 0)),
            pl.BlockSpec((RB, 1), lambda i: (i, 0)),
        ],
        out_shape=[
            jax.ShapeDtypeStruct((N, F), jnp.float32),
            jax.ShapeDtypeStruct((N, 1), jnp.float32),
            jax.ShapeDtypeStruct((N, 1), jnp.float32),
        ],
    )(x, W_gcn, cntp)

    a1 = _agg_kernel(src_r, dst_r, y)[:, :N, :]

    h1, r = pl.pallas_call(
        _k4_body,
        grid=(NRB,),
        in_specs=[
            pl.BlockSpec((NC, RB, F), lambda i: (0, i, 0)),
            pl.BlockSpec((RB, F), lambda i: (i, 0)),
            pl.BlockSpec((RB, 1), lambda i: (i, 0)),
            pl.BlockSpec((1, F), lambda i: (0, 0)),
            pl.BlockSpec((F, F), lambda i: (0, 0)),
            pl.BlockSpec((1, F), lambda i: (0, 0)),
        ],
        out_specs=[
            pl.BlockSpec((RB, F), lambda i: (i, 0)),
            pl.BlockSpec((RB, F), lambda i: (i, 0)),
        ],
        out_shape=[
            jax.ShapeDtypeStruct((N, F), jnp.float32),
            jax.ShapeDtypeStruct((N, F), jnp.float32),
        ],
    )(a1, y, dinv, bg, W_sage_r, bl)

    a2 = _agg_kernel(src_r, dst_r, h1)[:, :N, :]

    out = pl.pallas_call(
        _k6_body,
        grid=(NRB,),
        in_specs=[
            pl.BlockSpec((NC, RB, F), lambda i: (0, i, 0)),
            pl.BlockSpec((RB, 1), lambda i: (i, 0)),
            pl.BlockSpec((RB, F), lambda i: (i, 0)),
            pl.BlockSpec((F, F), lambda i: (0, 0)),
            pl.BlockSpec((1, 1, RB), lambda i: (i, 0, 0)),
            pl.BlockSpec((F, F), lambda i: (0, 0)),
            pl.BlockSpec((1, F), lambda i: (0, 0)),
            pl.BlockSpec((F, OUT), lambda i: (0, 0)),
            pl.BlockSpec((1, OUT), lambda i: (0, 0)),
        ],
        out_specs=pl.BlockSpec((G, OUT), lambda i: (0, 0)),
        out_shape=jax.ShapeDtypeStruct((G, OUT), jnp.float32),
        scratch_shapes=[pltpu.VMEM((G, F), jnp.float32)],
    )(a2, invc, r, W_sage_l, batch3d, W_fc1, bf1, W_fc2, bf2)

    return out
